# baseline (speedup 1.0000x reference)
import functools

import jax
import jax.numpy as jnp
from jax.experimental import pallas as pl


def _chamfer_body(m_ref, g_ref, o_ref, *, n, m):
    pts = m_ref[0]
    g = g_ref[0]
    hsq1 = -0.5 * jnp.sum(pts * pts, axis=1, keepdims=True)
    hsq2 = -0.5 * jnp.sum(g * g, axis=0, keepdims=True)
    inner = jax.lax.dot_general(
        pts, g, (((1,), (0,)), ((), ())),
        preferred_element_type=jnp.float32)
    u = inner + hsq2
    v = inner + hsq1
    h1 = jnp.max(u, axis=1) + hsq1[:, 0]
    h2 = jnp.max(v, axis=0) + hsq2[0, :]
    s1 = jnp.sum(jnp.sqrt(jnp.maximum(-2.0 * h1, 1e-9)))
    s2 = jnp.sum(jnp.sqrt(jnp.maximum(-2.0 * h2, 1e-9)))
    i = pl.program_id(0)
    row = jnp.concatenate(
        [jnp.broadcast_to(s1, (1, 1)), jnp.broadcast_to(s2, (1, 1))], axis=1)
    o_ref[pl.ds(i, 1), :] = row


def _chamfer_partials(means, gt_t):
    b, n, _ = means.shape
    _, _, m = gt_t.shape
    return pl.pallas_call(
        functools.partial(_chamfer_body, n=n, m=m),
        grid=(b,),
        in_specs=[
            pl.BlockSpec((1, n, 3), lambda i: (i, 0, 0)),
            pl.BlockSpec((1, 3, m), lambda i: (i, 0, 0)),
        ],
        out_specs=pl.BlockSpec((b, 2), lambda i: (0, 0)),
        out_shape=jax.ShapeDtypeStruct((b, 2), jnp.float32),
    )(means, gt_t)


def kernel(means, gt):
    b, n, _ = means.shape
    _, m, _ = gt.shape
    gt_t = gt.transpose(0, 2, 1)
    partial = _chamfer_partials(means, gt_t)
    s1 = jnp.sum(partial[:, 0]) / (b * n)
    s2 = jnp.sum(partial[:, 1]) / (b * m)
    return (s1 + s2) * 0.5 * 1000.0


# speedup vs baseline: 1.0042x; 1.0042x over previous
import functools

import jax
import jax.numpy as jnp
from jax.experimental import pallas as pl


def _chamfer_body(m_ref, g_ref, o_ref, *, n, m):
    pts = m_ref[0]
    g = g_ref[0]
    hsq1 = -0.5 * jnp.sum(pts * pts, axis=1, keepdims=True)
    hsq2 = -0.5 * jnp.sum(g * g, axis=0, keepdims=True)
    inner = jax.lax.dot_general(
        pts, g, (((1,), (0,)), ((), ())),
        preferred_element_type=jnp.float32)
    h = (inner + hsq1) + hsq2
    h1 = jnp.max(h, axis=1)
    h2 = jnp.max(h, axis=0)
    s1 = jnp.sum(jnp.sqrt(jnp.maximum(-2.0 * h1, 1e-9)))
    s2 = jnp.sum(jnp.sqrt(jnp.maximum(-2.0 * h2, 1e-9)))
    i = pl.program_id(0)
    row = jnp.concatenate(
        [jnp.broadcast_to(s1, (1, 1)), jnp.broadcast_to(s2, (1, 1))], axis=1)
    o_ref[pl.ds(i, 1), :] = row


def _chamfer_partials(means, gt_t):
    b, n, _ = means.shape
    _, _, m = gt_t.shape
    return pl.pallas_call(
        functools.partial(_chamfer_body, n=n, m=m),
        grid=(b,),
        in_specs=[
            pl.BlockSpec((1, n, 3), lambda i: (i, 0, 0)),
            pl.BlockSpec((1, 3, m), lambda i: (i, 0, 0)),
        ],
        out_specs=pl.BlockSpec((b, 2), lambda i: (0, 0)),
        out_shape=jax.ShapeDtypeStruct((b, 2), jnp.float32),
    )(means, gt_t)


def kernel(means, gt):
    b, n, _ = means.shape
    _, m, _ = gt.shape
    gt_t = gt.transpose(0, 2, 1)
    partial = _chamfer_partials(means, gt_t)
    s1 = jnp.sum(partial[:, 0]) / (b * n)
    s2 = jnp.sum(partial[:, 1]) / (b * m)
    return (s1 + s2) * 0.5 * 1000.0


# speedup vs baseline: 1.0227x; 1.0184x over previous
import functools

import jax
import jax.numpy as jnp
from jax.experimental import pallas as pl


def _chamfer_body(m_ref, g_ref, o_ref, *, n, m):
    pts = m_ref[0]
    hsq1 = -0.5 * jnp.sum(pts * pts, axis=1, keepdims=True)
    nchunks = 2
    mc = m // nchunks
    h1 = None
    s2 = jnp.float32(0.0)
    for j in range(nchunks):
        g = g_ref[0, :, pl.ds(j * mc, mc)]
        hsq2 = -0.5 * jnp.sum(g * g, axis=0, keepdims=True)
        inner = jax.lax.dot_general(
            pts, g, (((1,), (0,)), ((), ())),
            preferred_element_type=jnp.float32)
        h = (inner + hsq1) + hsq2
        hj = jnp.max(h, axis=1)
        h1 = hj if h1 is None else jnp.maximum(h1, hj)
        h2 = jnp.max(h, axis=0)
        s2 = s2 + jnp.sum(jnp.sqrt(jnp.maximum(-2.0 * h2, 1e-9)))
    s1 = jnp.sum(jnp.sqrt(jnp.maximum(-2.0 * h1, 1e-9)))
    i = pl.program_id(0)
    row = jnp.concatenate(
        [jnp.broadcast_to(s1, (1, 1)), jnp.broadcast_to(s2, (1, 1))], axis=1)
    o_ref[pl.ds(i, 1), :] = row


def _chamfer_partials(means, gt_t):
    b, n, _ = means.shape
    _, _, m = gt_t.shape
    return pl.pallas_call(
        functools.partial(_chamfer_body, n=n, m=m),
        grid=(b,),
        in_specs=[
            pl.BlockSpec((1, n, 3), lambda i: (i, 0, 0)),
            pl.BlockSpec((1, 3, m), lambda i: (i, 0, 0)),
        ],
        out_specs=pl.BlockSpec((b, 2), lambda i: (0, 0)),
        out_shape=jax.ShapeDtypeStruct((b, 2), jnp.float32),
    )(means, gt_t)


def kernel(means, gt):
    b, n, _ = means.shape
    _, m, _ = gt.shape
    gt_t = gt.transpose(0, 2, 1)
    partial = _chamfer_partials(means, gt_t)
    s1 = jnp.sum(partial[:, 0]) / (b * n)
    s2 = jnp.sum(partial[:, 1]) / (b * m)
    return (s1 + s2) * 0.5 * 1000.0
